# Initial kernel scaffold; baseline (speedup 1.0000x reference)
import jax
import jax.numpy as jnp
from jax.experimental import pallas as pl


def kernel(x, edge_index, W1_l, W1_r, b1, W2_l, W2_r, b2):
    raise NotImplementedError("write your pallas kernel here")


# speedup vs baseline: 4.3644x; 4.3644x over previous
import functools

import jax
import jax.numpy as jnp
from jax import lax
from jax.experimental import pallas as pl
from jax.experimental.pallas import tpu as pltpu
from jax.experimental.pallas import tpu_sc as plsc

N_NODES = 10000
N_EDGES = 320000
F = 128

NC = 2
NS = 16
CHUNK = 128
N_CHUNKS = -(-N_EDGES // (NC * NS * CHUNK))
E_PAD = NC * NS * N_CHUNKS * CHUNK
DUMMY = N_NODES
N_ACC = 10240
ROWS_PER_TILE = N_ACC // NS


def _sc_body(compute_deg, x_hbm, src_hbm, dst_hbm, agg_hbm, *rest):
    if compute_deg:
        (deg_hbm, src_c, dst_c, rows_v, degl_v, dacc_v,
         acc_sh, dstage_sh, sem) = rest
    else:
        (src_c, dst_c, rows_v, degl_v, dacc_v,
         acc_sh, dstage_sh, sem) = rest
        deg_hbm = None
    cid = lax.axis_index("c")
    sid = lax.axis_index("s")
    row0 = sid * ROWS_PER_TILE
    z16 = jnp.zeros((16,), jnp.float32)

    def zrow(i, _):
        for j in range(F // 16):
            rows_v[i, pl.ds(j * 16, 16)] = z16
        return 0
    lax.fori_loop(0, CHUNK, zrow, 0)
    for r in range(ROWS_PER_TILE // CHUNK):
        pltpu.sync_copy(rows_v, acc_sh.at[pl.ds(row0 + r * CHUNK, CHUNK)])
    if compute_deg:
        def zdeg(i, _):
            degl_v[pl.ds(i * 16, 16)] = z16
            return 0
        lax.fori_loop(0, N_ACC // 16, zdeg, 0)
    plsc.subcore_barrier()

    ones = jnp.full((16,), 1.0, jnp.float32)

    def chunk_body(j, _):
        pltpu.sync_copy(src_hbm.at[cid, sid, j], src_c)
        pltpu.sync_copy(dst_hbm.at[cid, sid, j], dst_c)
        pltpu.async_copy(x_hbm.at[src_c], rows_v, sem).wait()
        pltpu.sync_copy(rows_v, acc_sh.at[dst_c], add=True)
        if compute_deg:
            def grp(g, _):
                idx = dst_c[pl.ds(g * 16, 16)]
                plsc.addupdate_scatter(degl_v, [idx], ones)
                return 0
            lax.fori_loop(0, CHUNK // 16, grp, 0)
        return 0

    lax.fori_loop(0, N_CHUNKS, chunk_body, 0)

    if compute_deg:
        pltpu.sync_copy(degl_v, dstage_sh.at[pl.ds(sid * N_ACC, N_ACC)])
    plsc.subcore_barrier()

    pltpu.sync_copy(acc_sh.at[pl.ds(row0, ROWS_PER_TILE)],
                    agg_hbm.at[pl.ds(cid * N_ACC + row0, ROWS_PER_TILE)])
    if compute_deg:
        pltpu.sync_copy(dstage_sh.at[pl.ds(row0, ROWS_PER_TILE)],
                        degl_v.at[pl.ds(0, ROWS_PER_TILE)])
        for t in range(1, NS):
            pltpu.sync_copy(dstage_sh.at[pl.ds(t * N_ACC + row0, ROWS_PER_TILE)],
                            dacc_v)

            def addb(i, _):
                degl_v[pl.ds(i * 16, 16)] = (degl_v[pl.ds(i * 16, 16)]
                                             + dacc_v[pl.ds(i * 16, 16)])
                return 0
            lax.fori_loop(0, ROWS_PER_TILE // 16, addb, 0)
        pltpu.sync_copy(degl_v.at[pl.ds(0, ROWS_PER_TILE)],
                        deg_hbm.at[pl.ds(cid * N_ACC + row0, ROWS_PER_TILE)])


def _make_sc_pass(compute_deg):
    outs = [jax.ShapeDtypeStruct((NC * N_ACC, F), jnp.float32)]
    if compute_deg:
        outs.append(jax.ShapeDtypeStruct((NC * N_ACC,), jnp.float32))
    scratch = [
        pltpu.VMEM((CHUNK,), jnp.int32),
        pltpu.VMEM((CHUNK,), jnp.int32),
        pltpu.VMEM((CHUNK, F), jnp.float32),
        pltpu.VMEM((N_ACC,), jnp.float32),
        pltpu.VMEM((ROWS_PER_TILE,), jnp.float32),
        pltpu.VMEM_SHARED((N_ACC, F), jnp.float32),
        pltpu.VMEM_SHARED((NS * N_ACC,), jnp.float32),
        pltpu.SemaphoreType.DMA,
    ]
    mesh = plsc.VectorSubcoreMesh(core_axis_name="c", subcore_axis_name="s")
    return pl.kernel(
        functools.partial(_sc_body, compute_deg),
        out_type=outs,
        mesh=mesh,
        scratch_types=scratch,
        compiler_params=pltpu.CompilerParams(needs_layout_passes=False),
        name="sage_sc_segsum" + ("_deg" if compute_deg else ""),
    )


def _dense_body(agg_ref, deg_ref, x_ref, wl_ref, wr_ref, b_ref, out_ref):
    a = agg_ref[0] + agg_ref[1]
    d = deg_ref[0] + deg_ref[1]
    mean = a / jnp.maximum(d, 1.0)
    dn = (((1,), (0,)), ((), ()))
    h = lax.dot_general(mean, wl_ref[...], dn,
                        precision=lax.Precision.HIGHEST,
                        preferred_element_type=jnp.float32)
    h = h + lax.dot_general(x_ref[...], wr_ref[...], dn,
                            precision=lax.Precision.HIGHEST,
                            preferred_element_type=jnp.float32)
    out_ref[...] = jnp.maximum(h + b_ref[...], 0.0)


_BR = 1024


def _make_dense():
    return pl.pallas_call(
        _dense_body,
        grid=(-(-N_NODES // _BR),),
        in_specs=[
            pl.BlockSpec((NC, _BR, F), lambda i: (0, i, 0)),
            pl.BlockSpec((NC, _BR, 1), lambda i: (0, i, 0)),
            pl.BlockSpec((_BR, F), lambda i: (i, 0)),
            pl.BlockSpec((F, F), lambda i: (0, 0)),
            pl.BlockSpec((F, F), lambda i: (0, 0)),
            pl.BlockSpec((1, F), lambda i: (0, 0)),
        ],
        out_specs=pl.BlockSpec((_BR, F), lambda i: (i, 0)),
        out_shape=jax.ShapeDtypeStruct((N_NODES, F), jnp.float32),
        name="sage_dense",
    )


_sc_pass_deg = _make_sc_pass(True)
_sc_pass = _make_sc_pass(False)
_dense = _make_dense()


def kernel(x, edge_index, W1_l, W1_r, b1, W2_l, W2_r, b2):
    pad = E_PAD - N_EDGES
    src = jnp.concatenate([edge_index[0], jnp.zeros((pad,), jnp.int32)])
    dst = jnp.concatenate([edge_index[1], jnp.full((pad,), DUMMY, jnp.int32)])
    src = src.reshape(NC, NS, N_CHUNKS, CHUNK)
    dst = dst.reshape(NC, NS, N_CHUNKS, CHUNK)

    agg1, deg = _sc_pass_deg(x, src, dst)
    agg1 = agg1.reshape(NC, N_ACC, F)
    deg = deg.reshape(NC, N_ACC, 1)
    h = _dense(agg1, deg, x, W1_l, W1_r, b1.reshape(1, F))
    (agg2,) = _sc_pass(h, src, dst)
    agg2 = agg2.reshape(NC, N_ACC, F)
    return _dense(agg2, deg, h, W2_l, W2_r, b2.reshape(1, F))
